# baseline (speedup 1.0000x reference)
import functools

import jax
import jax.numpy as jnp
from jax import lax
from jax.experimental import pallas as pl
from jax.experimental.pallas import tpu as pltpu
from jax.experimental.pallas import tpu_sc as plsc

_LANE = 128
_PITCH = 129
_UNROLL = 4


@functools.cache
def _build(n_b, n_h, d):
    info = plsc.get_sparse_core_info()
    nc, ns = info.num_cores, info.num_subcores
    nw = nc * ns
    n_bt = n_b // _LANE
    n_units = n_h * n_bt
    upw = n_units // nw
    n_dt = d // 8
    nq = d // 16
    mesh = plsc.VectorSubcoreMesh(core_axis_name="c", subcore_axis_name="s")

    @functools.partial(
        pl.kernel,
        mesh=mesh,
        compiler_params=pltpu.CompilerParams(
            use_tc_tiling_on_sc=False, needs_layout_passes=False),
        out_type=jax.ShapeDtypeStruct((n_units * d, _LANE), jnp.float32),
        scratch_types=[
            pltpu.VMEM((upw, _LANE), jnp.int32),
            pltpu.VMEM((2, _LANE, d), jnp.float32),
            pltpu.VMEM((2, d, _PITCH), jnp.float32),
            pltpu.SemaphoreType.DMA,
            pltpu.SemaphoreType.DMA,
            pltpu.SemaphoreType.DMA,
            pltpu.SemaphoreType.DMA,
        ],
    )
    def gather_kernel(table_hbm, idx_hbm, out_hbm, idx_v, gbuf, tbuf,
                      sg0, sg1, so0, so1):
        semg = (sg0, sg1)
        semo = (so0, so1)
        wid = lax.axis_index("s") * nc + lax.axis_index("c")
        pltpu.sync_copy(idx_hbm.at[pl.ds(wid * upw, upw)], idx_v)
        k0 = wid * upw
        rowv = [lax.iota(jnp.int32, 16) + 16 * q for q in range(nq)]

        def fire(u, b):
            pltpu.make_async_copy(
                table_hbm.at[idx_v.at[u]], gbuf.at[b], semg[b]).start()

        for b in range(2):
            fire(b, b)

        bt_shift = n_bt.bit_length() - 1
        assert n_bt == 1 << bt_shift

        def unit(u, b):
            k = k0 + u
            h = lax.shift_right_logical(k, bt_shift)
            bt = k & (n_bt - 1)
            row0 = (h * (n_dt * n_bt) + bt) * 8
            pltpu.make_async_copy(
                table_hbm.at[idx_v.at[u]], gbuf.at[b], semg[b]).wait()

            @pl.when(u >= 2)
            def _():
                pltpu.make_async_copy(
                    tbuf.at[b].at[:, pl.ds(0, _LANE)],
                    out_hbm.at[pl.ds(0, d)], semo[b]).wait()

            def trans(i, blv):
                for s in range(_UNROLL):
                    bl = i * _UNROLL + s
                    bv = blv + s
                    for q in range(nq):
                        vec = gbuf.at[b][bl, pl.ds(q * 16, 16)]
                        plsc.store_scatter(tbuf.at[b], [rowv[q], bv], vec)
                return blv + _UNROLL

            lax.fori_loop(0, _LANE // _UNROLL, trans,
                          lax.iota(jnp.int32, 16) * 0)
            for dt in range(n_dt):
                pltpu.make_async_copy(
                    tbuf.at[b].at[pl.ds(dt * 8, 8), pl.ds(0, _LANE)],
                    out_hbm.at[pl.ds(row0 + dt * (n_bt * 8), 8)],
                    semo[b]).start()

            @pl.when(u + 2 < upw)
            def _():
                fire(u + 2, b)

        def step(i, carry):
            for b in range(2):
                unit(i * 2 + b, b)
            return carry

        lax.fori_loop(0, upw // 2, step, None)
        for b in range(2):
            pltpu.make_async_copy(
                tbuf.at[b].at[:, pl.ds(0, _LANE)],
                out_hbm.at[pl.ds(0, d)], semo[b]).wait()

    return gather_kernel


def kernel(indices, in_embeddings):
    n_b, n_h = indices.shape
    _, d = in_embeddings.shape
    n_bt = n_b // _LANE
    idx = indices.T.reshape(n_h * n_bt, _LANE).astype(jnp.int32)
    out2 = _build(n_b, n_h, d)(in_embeddings, idx)
    out5 = out2.reshape(n_h, d // 8, n_bt, 8, _LANE)
    return out5.transpose(2, 4, 0, 1, 3).reshape(n_b, n_h, d)


# speedup vs baseline: 1.2514x; 1.2514x over previous
import functools

import jax
import jax.numpy as jnp
from jax import lax
from jax.experimental import pallas as pl
from jax.experimental.pallas import tpu as pltpu
from jax.experimental.pallas import tpu_sc as plsc

_LANE = 128
_PITCH = 129
_UNROLL = 4


@functools.cache
def _build(n_b, n_h, d):
    info = plsc.get_sparse_core_info()
    nc, ns = info.num_cores, info.num_subcores
    nw = nc * ns
    n_bt = n_b // _LANE
    n_units = n_h * n_bt
    upw = n_units // nw
    n_dt = d // 8
    nq = d // 16
    mesh = plsc.VectorSubcoreMesh(core_axis_name="c", subcore_axis_name="s")

    @functools.partial(
        pl.kernel,
        mesh=mesh,
        compiler_params=pltpu.CompilerParams(
            use_tc_tiling_on_sc=False, needs_layout_passes=False),
        out_type=jax.ShapeDtypeStruct((n_units * d, _LANE), jnp.float32),
        scratch_types=[
            pltpu.VMEM((upw, _LANE), jnp.int32),
            pltpu.VMEM((2, _LANE, d), jnp.float32),
            pltpu.VMEM((2, d, _PITCH), jnp.float32),
            pltpu.SemaphoreType.DMA,
            pltpu.SemaphoreType.DMA,
            pltpu.SemaphoreType.DMA,
            pltpu.SemaphoreType.DMA,
        ],
    )
    def gather_kernel(table_hbm, idx_hbm, out_hbm, idx_v, gbuf, tbuf,
                      sg0, sg1, so0, so1):
        semg = (sg0, sg1)
        semo = (so0, so1)
        wid = lax.axis_index("s") * nc + lax.axis_index("c")
        pltpu.sync_copy(idx_hbm.at[pl.ds(wid * upw, upw)], idx_v)
        k0 = wid * upw
        rowv = [lax.iota(jnp.int32, 16) + 16 * q for q in range(nq)]

        def fire(u, b):
            pltpu.make_async_copy(
                table_hbm.at[idx_v.at[u]], gbuf.at[b], semg[b]).start()

        for b in range(2):
            fire(b, b)

        bt_shift = n_bt.bit_length() - 1
        assert n_bt == 1 << bt_shift

        def unit(u, b):
            k = k0 + u
            h = lax.shift_right_logical(k, bt_shift)
            bt = k & (n_bt - 1)
            row0 = (h * (n_dt * n_bt) + bt) * 8
            pltpu.make_async_copy(
                table_hbm.at[idx_v.at[u]], gbuf.at[b], semg[b]).wait()

            @pl.when(u >= 2)
            def _():
                pltpu.make_async_copy(
                    tbuf.at[b].at[:, pl.ds(0, _LANE)],
                    out_hbm.at[pl.ds(0, d)], semo[b]).wait()

            @plsc.parallel_loop(0, _LANE, 1, unroll=_UNROLL)
            def trans(bl):
                bv = jnp.full((16,), bl, jnp.int32)
                for q in range(nq):
                    vec = gbuf.at[b][bl, pl.ds(q * 16, 16)]
                    plsc.store_scatter(tbuf.at[b], [rowv[q], bv], vec)
            for dt in range(n_dt):
                pltpu.make_async_copy(
                    tbuf.at[b].at[pl.ds(dt * 8, 8), pl.ds(0, _LANE)],
                    out_hbm.at[pl.ds(row0 + dt * (n_bt * 8), 8)],
                    semo[b]).start()

            @pl.when(u + 2 < upw)
            def _():
                fire(u + 2, b)

        def step(i, carry):
            for b in range(2):
                unit(i * 2 + b, b)
            return carry

        lax.fori_loop(0, upw // 2, step, None)
        for b in range(2):
            pltpu.make_async_copy(
                tbuf.at[b].at[:, pl.ds(0, _LANE)],
                out_hbm.at[pl.ds(0, d)], semo[b]).wait()

    return gather_kernel


def kernel(indices, in_embeddings):
    n_b, n_h = indices.shape
    _, d = in_embeddings.shape
    n_bt = n_b // _LANE
    idx = indices.T.reshape(n_h * n_bt, _LANE).astype(jnp.int32)
    out2 = _build(n_b, n_h, d)(in_embeddings, idx)
    out5 = out2.reshape(n_h, d // 8, n_bt, 8, _LANE)
    return out5.transpose(2, 4, 0, 1, 3).reshape(n_b, n_h, d)


# speedup vs baseline: 1.2548x; 1.0028x over previous
import functools

import jax
import jax.numpy as jnp
from jax import lax
from jax.experimental import pallas as pl
from jax.experimental.pallas import tpu as pltpu
from jax.experimental.pallas import tpu_sc as plsc

_LANE = 128
_PITCH = 129
_UNROLL = 8


@functools.cache
def _build(n_b, n_h, d):
    info = plsc.get_sparse_core_info()
    nc, ns = info.num_cores, info.num_subcores
    nw = nc * ns
    n_bt = n_b // _LANE
    n_units = n_h * n_bt
    upw = n_units // nw
    n_dt = d // 8
    nq = d // 16
    mesh = plsc.VectorSubcoreMesh(core_axis_name="c", subcore_axis_name="s")

    @functools.partial(
        pl.kernel,
        mesh=mesh,
        compiler_params=pltpu.CompilerParams(
            use_tc_tiling_on_sc=False, needs_layout_passes=False),
        out_type=jax.ShapeDtypeStruct((n_units * d, _LANE), jnp.float32),
        scratch_types=[
            pltpu.VMEM((upw, _LANE), jnp.int32),
            pltpu.VMEM((2, _LANE, d), jnp.float32),
            pltpu.VMEM((2, d, _PITCH), jnp.float32),
            pltpu.SemaphoreType.DMA,
            pltpu.SemaphoreType.DMA,
            pltpu.SemaphoreType.DMA,
            pltpu.SemaphoreType.DMA,
        ],
    )
    def gather_kernel(table_hbm, idx_hbm, out_hbm, idx_v, gbuf, tbuf,
                      sg0, sg1, so0, so1):
        semg = (sg0, sg1)
        semo = (so0, so1)
        wid = lax.axis_index("s") * nc + lax.axis_index("c")
        pltpu.sync_copy(idx_hbm.at[pl.ds(wid * upw, upw)], idx_v)
        k0 = wid * upw
        rowv = [lax.iota(jnp.int32, 16) + 16 * q for q in range(nq)]

        def fire(u, b):
            pltpu.make_async_copy(
                table_hbm.at[idx_v.at[u]], gbuf.at[b], semg[b]).start()

        for b in range(2):
            fire(b, b)

        bt_shift = n_bt.bit_length() - 1
        assert n_bt == 1 << bt_shift

        def unit(u, b):
            k = k0 + u
            h = lax.shift_right_logical(k, bt_shift)
            bt = k & (n_bt - 1)
            row0 = (h * (n_dt * n_bt) + bt) * 8
            pltpu.make_async_copy(
                table_hbm.at[idx_v.at[u]], gbuf.at[b], semg[b]).wait()

            @pl.when(u >= 2)
            def _():
                pltpu.make_async_copy(
                    tbuf.at[b].at[:, pl.ds(0, _LANE)],
                    out_hbm.at[pl.ds(0, d)], semo[b]).wait()

            @plsc.parallel_loop(0, _LANE, 1, unroll=_UNROLL)
            def trans(bl):
                bv = jnp.full((16,), bl, jnp.int32)
                for q in range(nq):
                    vec = gbuf.at[b][bl, pl.ds(q * 16, 16)]
                    plsc.store_scatter(tbuf.at[b], [rowv[q], bv], vec)
            for dt in range(n_dt):
                pltpu.make_async_copy(
                    tbuf.at[b].at[pl.ds(dt * 8, 8), pl.ds(0, _LANE)],
                    out_hbm.at[pl.ds(row0 + dt * (n_bt * 8), 8)],
                    semo[b]).start()

            @pl.when(u + 2 < upw)
            def _():
                fire(u + 2, b)

        def step(i, carry):
            for b in range(2):
                unit(i * 2 + b, b)
            return carry

        lax.fori_loop(0, upw // 2, step, None)
        for b in range(2):
            pltpu.make_async_copy(
                tbuf.at[b].at[:, pl.ds(0, _LANE)],
                out_hbm.at[pl.ds(0, d)], semo[b]).wait()

    return gather_kernel


def kernel(indices, in_embeddings):
    n_b, n_h = indices.shape
    _, d = in_embeddings.shape
    n_bt = n_b // _LANE
    idx = indices.T.reshape(n_h * n_bt, _LANE).astype(jnp.int32)
    out2 = _build(n_b, n_h, d)(in_embeddings, idx)
    out5 = out2.reshape(n_h, d // 8, n_bt, 8, _LANE)
    return out5.transpose(2, 4, 0, 1, 3).reshape(n_b, n_h, d)
